# baseline (speedup 1.0000x reference)
import jax
import jax.numpy as jnp
from jax.experimental import pallas as pl


def _add_kernel(p_ref, t_ref, o_ref):
    o_ref[...] = p_ref[...] + t_ref[...]


def kernel(encoded_patches, position_table):
    B, P, D = encoded_patches.shape
    return pl.pallas_call(
        _add_kernel,
        grid=(B,),
        in_specs=[
            pl.BlockSpec((1, P, D), lambda i: (i, 0, 0)),
            pl.BlockSpec((P, D), lambda i: (0, 0)),
        ],
        out_specs=pl.BlockSpec((1, P, D), lambda i: (i, 0, 0)),
        out_shape=jax.ShapeDtypeStruct((B, P, D), encoded_patches.dtype),
    )(encoded_patches, position_table)


# speedup vs baseline: 1.1834x; 1.1834x over previous
import jax
import jax.numpy as jnp
from jax.experimental import pallas as pl


def _add_kernel(p_ref, t_ref, o_ref):
    o_ref[...] = p_ref[...] + t_ref[...]


def kernel(encoded_patches, position_table):
    B, P, D = encoded_patches.shape
    BB = 4
    return pl.pallas_call(
        _add_kernel,
        grid=(B // BB,),
        in_specs=[
            pl.BlockSpec((BB, P, D), lambda i: (i, 0, 0)),
            pl.BlockSpec((P, D), lambda i: (0, 0)),
        ],
        out_specs=pl.BlockSpec((BB, P, D), lambda i: (i, 0, 0)),
        out_shape=jax.ShapeDtypeStruct((B, P, D), encoded_patches.dtype),
    )(encoded_patches, position_table)


# speedup vs baseline: 1.2015x; 1.0153x over previous
import jax
import jax.numpy as jnp
from jax.experimental import pallas as pl


def _add_kernel(p_ref, t_ref, o_ref):
    o_ref[...] = p_ref[...] + t_ref[...]


def kernel(encoded_patches, position_table):
    B, P, D = encoded_patches.shape
    BB = 8
    return pl.pallas_call(
        _add_kernel,
        grid=(B // BB,),
        in_specs=[
            pl.BlockSpec((BB, P, D), lambda i: (i, 0, 0)),
            pl.BlockSpec((P, D), lambda i: (0, 0)),
        ],
        out_specs=pl.BlockSpec((BB, P, D), lambda i: (i, 0, 0)),
        out_shape=jax.ShapeDtypeStruct((B, P, D), encoded_patches.dtype),
    )(encoded_patches, position_table)
